# baseline (speedup 1.0000x reference)
import functools

import jax
import jax.numpy as jnp
from jax import lax
from jax.experimental import pallas as pl
from jax.experimental.pallas import tpu as pltpu
from jax.experimental.pallas import tpu_sc as plsc

_NC = 2
_NS = 16
_NW = _NC * _NS

_CL = 100
_NBUF = 4
_LANES = 16
_UNROLL = 10


@functools.lru_cache(maxsize=None)
def _build(B, L, D, V):
    CPR = L // _CL
    RPW = B // _NW
    CPW = RPW * CPR
    NI = CPW // _NBUF
    NG = D // _LANES
    scale = 1.0 / L

    mesh = plsc.VectorSubcoreMesh(core_axis_name="c", subcore_axis_name="s")

    @functools.partial(
        pl.kernel,
        out_type=jax.ShapeDtypeStruct((B, D), jnp.float32),
        mesh=mesh,
        compiler_params=pltpu.CompilerParams(use_tc_tiling_on_sc=False),
        scratch_types=[
            pltpu.VMEM((CPW, _CL), jnp.int32),
            pltpu.VMEM((_NBUF, _CL, D), jnp.float32),
            pltpu.VMEM((RPW, D), jnp.float32),
            pltpu.SemaphoreType.DMA,
            pltpu.SemaphoreType.DMA,
            pltpu.SemaphoreType.DMA,
            pltpu.SemaphoreType.DMA,
        ],
    )
    def swem(idx_hbm, table_hbm, out_hbm, idx_v, buf_v, out_v, *sems):
        wid = lax.axis_index("s") * _NC + lax.axis_index("c")
        cbase = wid * CPW
        rbase = wid * RPW

        pltpu.sync_copy(idx_hbm.at[pl.ds(cbase, CPW)], idx_v)

        def gather(c, s):
            pltpu.async_copy(table_hbm.at[idx_v.at[c]], buf_v.at[s], sems[s])

        def gwait(c, s):
            pltpu.make_async_copy(
                table_hbm.at[idx_v.at[c]], buf_v.at[s], sems[s]
            ).wait()

        for s in range(_NBUF):
            gather(s, s)

        def accum(s, accs):
            buf = buf_v.at[s]

            def body(k, accs):
                accs = list(accs)
                for u in range(_UNROLL):
                    r = k * _UNROLL + u
                    for g in range(NG):
                        slot = (u % 2) * NG + g
                        accs[slot] = accs[slot] + buf[r, pl.ds(_LANES * g, _LANES)]
                return tuple(accs)

            return lax.fori_loop(0, _CL // _UNROLL, body, accs)

        def iteration(it, carry):
            for half in range(2):
                row = 2 * it + half
                s0 = 2 * half
                c0 = 4 * it + s0
                accs = tuple(
                    jnp.zeros((_LANES,), jnp.float32) for _ in range(2 * NG)
                )
                for k in range(CPR):
                    s = s0 + k
                    c = c0 + k
                    gwait(c, s)
                    accs = accum(s, accs)

                    @pl.when(it < NI - 1)
                    def _():
                        gather(c + _NBUF, s)

                for g in range(NG):
                    out_v[row, pl.ds(_LANES * g, _LANES)] = (
                        accs[g] + accs[NG + g]
                    ) * scale
            return carry

        lax.fori_loop(0, NI, iteration, 0)

        pltpu.sync_copy(out_v, out_hbm.at[pl.ds(rbase, RPW)])

    return swem


def kernel(input, table):
    B, L = input.shape
    V, D = table.shape
    idx = input.astype(jnp.int32).reshape(B * L // _CL, _CL)
    return _build(B, L, D, V)(idx, table)


# speedup vs baseline: 1.0221x; 1.0221x over previous
import functools

import jax
import jax.numpy as jnp
from jax import lax
from jax.experimental import pallas as pl
from jax.experimental.pallas import tpu as pltpu
from jax.experimental.pallas import tpu_sc as plsc

_NC = 2
_NS = 16
_NW = _NC * _NS

_CL = 100
_NBUF = 8
_LANES = 16
_UNROLL = 10


@functools.lru_cache(maxsize=None)
def _build(B, L, D, V):
    CPR = L // _CL
    RPW = B // _NW
    CPW = RPW * CPR
    RPI = _NBUF // CPR
    NI = CPW // _NBUF
    NG = D // _LANES
    scale = 1.0 / L

    mesh = plsc.VectorSubcoreMesh(core_axis_name="c", subcore_axis_name="s")

    @functools.partial(
        pl.kernel,
        out_type=jax.ShapeDtypeStruct((B, D), jnp.float32),
        mesh=mesh,
        compiler_params=pltpu.CompilerParams(use_tc_tiling_on_sc=False),
        scratch_types=[
            pltpu.VMEM((CPW, _CL), jnp.int32),
            pltpu.VMEM((_NBUF, _CL, D), jnp.float32),
            pltpu.VMEM((RPW, D), jnp.float32),
        ]
        + [pltpu.SemaphoreType.DMA] * _NBUF,
    )
    def swem(idx_hbm, table_hbm, out_hbm, idx_v, buf_v, out_v, *sems):
        wid = lax.axis_index("s") * _NC + lax.axis_index("c")
        cbase = wid * CPW
        rbase = wid * RPW

        pltpu.sync_copy(idx_hbm.at[pl.ds(cbase, CPW)], idx_v)

        def gather(c, s):
            pltpu.async_copy(table_hbm.at[idx_v.at[c]], buf_v.at[s], sems[s])

        def gwait(c, s):
            pltpu.make_async_copy(
                table_hbm.at[idx_v.at[c]], buf_v.at[s], sems[s]
            ).wait()

        for s in range(_NBUF):
            gather(s, s)

        def accum(s, accs):
            buf = buf_v.at[s]

            def body(k, accs):
                accs = list(accs)
                for u in range(_UNROLL):
                    r = k * _UNROLL + u
                    for g in range(NG):
                        slot = (u % 2) * NG + g
                        accs[slot] = accs[slot] + buf[r, pl.ds(_LANES * g, _LANES)]
                return tuple(accs)

            return lax.fori_loop(0, _CL // _UNROLL, body, accs)

        def iteration(it, carry):
            for j in range(RPI):
                row = RPI * it + j
                accs = tuple(
                    jnp.zeros((_LANES,), jnp.float32) for _ in range(2 * NG)
                )
                for k in range(CPR):
                    s = CPR * j + k
                    c = _NBUF * it + s
                    gwait(c, s)
                    accs = accum(s, accs)

                    @pl.when(it < NI - 1)
                    def _():
                        gather(c + _NBUF, s)

                for g in range(NG):
                    out_v[row, pl.ds(_LANES * g, _LANES)] = (
                        accs[g] + accs[NG + g]
                    ) * scale
            return carry

        lax.fori_loop(0, NI, iteration, 0)

        pltpu.sync_copy(out_v, out_hbm.at[pl.ds(rbase, RPW)])

    return swem


def kernel(input, table):
    B, L = input.shape
    V, D = table.shape
    idx = input.astype(jnp.int32).reshape(B * L // _CL, _CL)
    return _build(B, L, D, V)(idx, table)
